# baseline (speedup 1.0000x reference)
import functools

import jax
import jax.numpy as jnp
from jax import lax
from jax.experimental import pallas as pl
from jax.experimental.pallas import tpu as pltpu
from jax.experimental.pallas import tpu_sc as plsc

DIM_ = 64
NC_ = 2
NS_ = 16
NW_ = NC_ * NS_
CHUNK_ = 128


@functools.partial(jax.jit, static_argnames=("b_per_w", "n_chunks"))
def _gather_call(idx_flat, table, *, b_per_w, n_chunks):
    B = idx_flat.shape[0]
    mesh = plsc.VectorSubcoreMesh(core_axis_name="c", subcore_axis_name="s")

    @functools.partial(
        pl.kernel,
        mesh=mesh,
        out_type=jax.ShapeDtypeStruct((B, DIM_), jnp.float32),
        scratch_types=[
            pltpu.VMEM((2, CHUNK_), jnp.int32),
            pltpu.VMEM((2, CHUNK_, DIM_), jnp.float32),
            pltpu.SemaphoreType.DMA,
            pltpu.SemaphoreType.DMA,
        ],
        compiler_params=pltpu.CompilerParams(use_tc_tiling_on_sc=False),
    )
    def k(idx_hbm, table_hbm, out_hbm, idx_v, rows_v, sem0, sem1):
        wid = lax.axis_index("s") * NC_ + lax.axis_index("c")
        base = wid * b_per_w
        sems = (sem0, sem1)

        def start(g, buf):
            pltpu.sync_copy(idx_hbm.at[pl.ds(base + g * CHUNK_, CHUNK_)],
                            idx_v.at[buf])
            pltpu.async_copy(table_hbm.at[idx_v.at[buf]], rows_v.at[buf],
                             sems[buf])

        def finish(g, buf):
            pltpu.make_async_copy(table_hbm.at[idx_v.at[buf]], rows_v.at[buf],
                                  sems[buf]).wait()
            pltpu.sync_copy(rows_v.at[buf],
                            out_hbm.at[pl.ds(base + g * CHUNK_, CHUNK_)])

        n_pairs = n_chunks // 2
        start(0, 0)

        def pair(p, carry):
            g0 = 2 * p
            start(g0 + 1, 1)
            finish(g0, 0)

            @pl.when(p + 1 < n_pairs)
            def _():
                start(g0 + 2, 0)

            finish(g0 + 1, 1)
            return carry

        lax.fori_loop(0, n_pairs, pair, 0)

    return k(idx_flat, table)


def kernel(x, emb_weight):
    B = x.shape[0] * x.shape[1]
    b_per_w = B // NW_
    n_chunks = b_per_w // CHUNK_
    idx_flat = x.reshape(B).astype(jnp.int32)
    out = _gather_call(idx_flat, emb_weight, b_per_w=b_per_w, n_chunks=n_chunks)
    return out.reshape(x.shape[0], x.shape[1], DIM_)


# speedup vs baseline: 1.0595x; 1.0595x over previous
import functools

import jax
import jax.numpy as jnp
from jax import lax
from jax.experimental import pallas as pl
from jax.experimental.pallas import tpu as pltpu
from jax.experimental.pallas import tpu_sc as plsc

DIM_ = 64
NC_ = 2
NS_ = 16
NW_ = NC_ * NS_
CHUNK_ = 128
GRP_ = 20
NBUF_ = 10
DEPTH_ = 5


@functools.partial(jax.jit, static_argnames=("n_groups",))
def _gather_call(idx4, table, *, n_groups):
    n_chunks = n_groups * GRP_
    B = NW_ * n_chunks * CHUNK_
    mesh = plsc.VectorSubcoreMesh(core_axis_name="c", subcore_axis_name="s")

    sem_types = [pltpu.SemaphoreType.DMA] * (2 * NBUF_ + 1)

    @functools.partial(
        pl.kernel,
        mesh=mesh,
        out_type=jax.ShapeDtypeStruct((B, DIM_), jnp.float32),
        scratch_types=[
            pltpu.VMEM((2, GRP_, CHUNK_), jnp.int32),
            pltpu.VMEM((NBUF_, CHUNK_, DIM_), jnp.float32),
        ] + sem_types,
        compiler_params=pltpu.CompilerParams(use_tc_tiling_on_sc=False),
    )
    def k(idx_hbm, table_hbm, out_hbm, idx_v, rows_v, *sems):
        gsem = sems[:NBUF_]
        ssem = sems[NBUF_:2 * NBUF_]
        isem = sems[2 * NBUF_:]
        wid = lax.axis_index("s") * NC_ + lax.axis_index("c")
        base = wid * n_chunks * CHUNK_

        def idx_copy(g, gb):
            return pltpu.make_async_copy(idx_hbm.at[wid, g], idx_v.at[gb],
                                         isem[0])

        def start_gather(gb, j, b):
            pltpu.async_copy(table_hbm.at[idx_v.at[gb, j]], rows_v.at[b],
                             gsem[b])

        def wait_gather(b):
            pltpu.make_async_copy(table_hbm.at[idx_v.at[0, 0]], rows_v.at[b],
                                  gsem[b]).wait()

        def start_store(s, b):
            pltpu.async_copy(rows_v.at[b],
                             out_hbm.at[pl.ds(base + s * CHUNK_, CHUNK_)],
                             ssem[b])

        def wait_store(b):
            pltpu.make_async_copy(rows_v.at[b],
                                  out_hbm.at[pl.ds(base, CHUNK_)],
                                  ssem[b]).wait()

        pltpu.sync_copy(idx_hbm.at[wid, 0], idx_v.at[0])
        for j in range(DEPTH_):
            start_gather(0, j, j % NBUF_)

        def group_body(g, carry):
            gb_cur = g % 2
            gb_nxt = (g + 1) % 2
            for j in range(GRP_):
                s = g * GRP_ + j
                b = j % NBUF_

                if j == 0:
                    @pl.when(g + 1 < n_groups)
                    def _():
                        idx_copy(g + 1, gb_nxt).start()

                wait_gather(b)
                start_store(s, b)

                nxt_j = j + DEPTH_
                b2 = nxt_j % NBUF_

                @pl.when(s + DEPTH_ >= NBUF_)
                def _():
                    wait_store(b2)

                if j == GRP_ - DEPTH_:
                    @pl.when(g + 1 < n_groups)
                    def _():
                        idx_copy(g + 1, gb_nxt).wait()

                if nxt_j < GRP_:
                    @pl.when(s + DEPTH_ < n_chunks)
                    def _():
                        start_gather(gb_cur, nxt_j, b2)
                else:
                    @pl.when(s + DEPTH_ < n_chunks)
                    def _():
                        start_gather(gb_nxt, nxt_j - GRP_, b2)
            return carry

        lax.fori_loop(0, n_groups, group_body, 0)

        for i in range(DEPTH_):
            wait_store((n_chunks - DEPTH_ + i) % NBUF_)

    return k(idx4, table)


def kernel(x, emb_weight):
    B = x.shape[0] * x.shape[1]
    n_groups = B // (NW_ * GRP_ * CHUNK_)
    idx4 = x.reshape(NW_, n_groups, GRP_, CHUNK_).astype(jnp.int32)
    out = _gather_call(idx4, emb_weight, n_groups=n_groups)
    return out.reshape(x.shape[0], x.shape[1], DIM_)
